# initial kernel scaffold (unmeasured)
import jax
import jax.numpy as jnp
from jax import lax
from jax.experimental import pallas as pl
from jax.experimental.pallas import tpu as pltpu

N_DEV = 16
B, S, H, Dh, Dr = 4, 256, 32, 128, 64
D = 4096
CH = S // N_DEV
NSLOTS = 2


def _allreduce_kv(kv):

    def chunk(ref, idx):
        return ref.at[:, :, pl.ds(idx * CH, CH), :]

    def body(kv_ref, out_ref, recv_buf, rs_send, rs_recv, ag_send, ag_recv,
             credit_rs, credit_ag):
        p = lax.axis_index("i")
        left = (p - 1) % N_DEV
        right = (p + 1) % N_DEV

        barrier_sem = pltpu.get_barrier_semaphore()
        for nbr in (left, right):
            pl.semaphore_signal(barrier_sem, inc=1, device_id=(nbr,),
                                device_id_type=pl.DeviceIdType.MESH)
        pl.semaphore_wait(barrier_sem, 2)

        out_ref[...] = kv_ref[...]

        for s in range(N_DEV - 1):
            slot = s % NSLOTS
            send_idx = (p - s) % N_DEV
            recv_idx = (p - s - 1) % N_DEV
            if s >= NSLOTS:
                pl.semaphore_wait(credit_rs, 1)
            rdma = pltpu.make_async_remote_copy(
                src_ref=chunk(out_ref, send_idx),
                dst_ref=recv_buf.at[slot],
                send_sem=rs_send.at[slot],
                recv_sem=rs_recv.at[slot],
                device_id=(right,),
                device_id_type=pl.DeviceIdType.MESH,
            )
            rdma.start()
            rdma.wait()
            out_ref[:, :, pl.ds(recv_idx * CH, CH), :] += recv_buf[slot]
            if s < (N_DEV - 1) - NSLOTS:
                pl.semaphore_signal(credit_rs, inc=1, device_id=(left,),
                                    device_id_type=pl.DeviceIdType.MESH)

        for s in range(N_DEV - 1):
            slot = s % NSLOTS
            send_idx = (p + 1 - s) % N_DEV
            if s >= NSLOTS:
                pl.semaphore_wait(credit_ag, 1)
            rdma = pltpu.make_async_remote_copy(
                src_ref=chunk(out_ref, send_idx),
                dst_ref=chunk(out_ref, send_idx),
                send_sem=ag_send.at[slot],
                recv_sem=ag_recv.at[slot],
                device_id=(right,),
                device_id_type=pl.DeviceIdType.MESH,
            )
            rdma.start()
            rdma.wait()
            if s < (N_DEV - 1) - NSLOTS:
                pl.semaphore_signal(credit_ag, inc=1, device_id=(left,),
                                    device_id_type=pl.DeviceIdType.MESH)

    return pl.pallas_call(
        body,
        out_shape=jax.ShapeDtypeStruct((2, B, S, D), kv.dtype),
        in_specs=[pl.BlockSpec(memory_space=pltpu.VMEM)],
        out_specs=pl.BlockSpec(memory_space=pltpu.VMEM),
        scratch_shapes=[
            pltpu.VMEM((NSLOTS, 2, B, CH, D), kv.dtype),
            pltpu.SemaphoreType.DMA((NSLOTS,)),
            pltpu.SemaphoreType.DMA((NSLOTS,)),
            pltpu.SemaphoreType.DMA((NSLOTS,)),
            pltpu.SemaphoreType.DMA((NSLOTS,)),
            pltpu.SemaphoreType.REGULAR,
            pltpu.SemaphoreType.REGULAR,
        ],
        compiler_params=pltpu.CompilerParams(collective_id=0),
    )(kv)


def kernel(x, Wdkv, Wuk, Wuv, Wq, Wqr, Wkr, Wo):
    c = x @ Wdkv
    Kp = c @ Wuk
    Vp = c @ Wuv
    kv = _allreduce_kv(jnp.stack([Kp, Vp]))
    K = kv[0].reshape(B, S, H, Dh)
    V = kv[1].reshape(B, S, H, Dh)

    Q = (x @ Wq).reshape(B, S, H, Dh)
    Qr = (x @ Wqr).reshape(B, S, H, Dr)
    Kr = (x @ Wkr).reshape(B, S, 1, Dr)

    scale = (Dh + Dr) ** -0.5
    scores = (jnp.einsum("bshd,bthd->bhst", Q, K)
              + jnp.einsum("bshd,bthd->bhst", Qr,
                           jnp.broadcast_to(Kr, (B, S, H, Dr)))) * scale
    m = scores.max(-1, keepdims=True)
    P = jnp.exp(scores - m)
    P = P / P.sum(-1, keepdims=True)
    O = jnp.einsum("bhst,bthd->bshd", P, V).reshape(B, S, H * Dh)
    return (O @ Wo).astype(jnp.float32)


# baseline (device time: 996827 ns/iter reference)
import jax
import jax.numpy as jnp
from jax import lax
from jax.experimental import pallas as pl
from jax.experimental.pallas import tpu as pltpu

N_DEV = 16
B, S, H, Dh, Dr = 4, 256, 32, 128, 64
D = 4096
CH = S // N_DEV
NSLOTS = 2


def _allreduce_kv(kv):

    def chunk(ref, idx):
        return ref.at[:, :, pl.ds(idx * CH, CH), :]

    def body(kv_ref, out_ref, recv_buf, rs_send, rs_recv, ag_send, ag_recv,
             credit_rs, credit_ag, copy_sem):
        p = lax.axis_index("i")
        left = (p - 1) % N_DEV
        right = (p + 1) % N_DEV

        cp = pltpu.make_async_copy(kv_ref, out_ref, copy_sem)
        cp.start()

        barrier_sem = pltpu.get_barrier_semaphore()
        for nbr in (left, right):
            pl.semaphore_signal(barrier_sem, inc=1, device_id=(nbr,),
                                device_id_type=pl.DeviceIdType.MESH)
        pl.semaphore_wait(barrier_sem, 2)
        cp.wait()

        for s in range(N_DEV - 1):
            slot = s % NSLOTS
            send_idx = (p - s) % N_DEV
            recv_idx = (p - s - 1) % N_DEV
            if s >= NSLOTS:
                pl.semaphore_wait(credit_rs, 1)
            rdma = pltpu.make_async_remote_copy(
                src_ref=chunk(out_ref, send_idx),
                dst_ref=recv_buf.at[slot],
                send_sem=rs_send.at[slot],
                recv_sem=rs_recv.at[slot],
                device_id=(right,),
                device_id_type=pl.DeviceIdType.MESH,
            )
            rdma.start()
            rdma.wait()
            out_ref[:, :, pl.ds(recv_idx * CH, CH), :] += recv_buf[slot]
            if s < (N_DEV - 1) - NSLOTS:
                pl.semaphore_signal(credit_rs, inc=1, device_id=(left,),
                                    device_id_type=pl.DeviceIdType.MESH)

        for s in range(N_DEV - 1):
            slot = s % NSLOTS
            send_idx = (p + 1 - s) % N_DEV
            if s >= NSLOTS:
                pl.semaphore_wait(credit_ag, 1)
            rdma = pltpu.make_async_remote_copy(
                src_ref=chunk(out_ref, send_idx),
                dst_ref=chunk(out_ref, send_idx),
                send_sem=ag_send.at[slot],
                recv_sem=ag_recv.at[slot],
                device_id=(right,),
                device_id_type=pl.DeviceIdType.MESH,
            )
            rdma.start()
            rdma.wait()
            if s < (N_DEV - 1) - NSLOTS:
                pl.semaphore_signal(credit_ag, inc=1, device_id=(left,),
                                    device_id_type=pl.DeviceIdType.MESH)

    return pl.pallas_call(
        body,
        out_shape=jax.ShapeDtypeStruct((2, B, S, D), kv.dtype),
        in_specs=[pl.BlockSpec(memory_space=pltpu.MemorySpace.HBM)],
        out_specs=pl.BlockSpec(memory_space=pltpu.VMEM),
        scratch_shapes=[
            pltpu.VMEM((NSLOTS, 2, B, CH, D), kv.dtype),
            pltpu.SemaphoreType.DMA((NSLOTS,)),
            pltpu.SemaphoreType.DMA((NSLOTS,)),
            pltpu.SemaphoreType.DMA((NSLOTS,)),
            pltpu.SemaphoreType.DMA((NSLOTS,)),
            pltpu.SemaphoreType.REGULAR,
            pltpu.SemaphoreType.REGULAR,
            pltpu.SemaphoreType.DMA,
        ],
        compiler_params=pltpu.CompilerParams(
            collective_id=0, vmem_limit_bytes=63 * 1024 * 1024),
    )(kv)


def kernel(x, Wdkv, Wuk, Wuv, Wq, Wqr, Wkr, Wo):
    c = x @ Wdkv
    Kp = c @ Wuk
    Vp = c @ Wuv
    kv = _allreduce_kv(jnp.stack([Kp, Vp]))
    K = kv[0].reshape(B, S, H, Dh)
    V = kv[1].reshape(B, S, H, Dh)

    Q = (x @ Wq).reshape(B, S, H, Dh)
    Qr = (x @ Wqr).reshape(B, S, H, Dr)
    Kr = (x @ Wkr).reshape(B, S, 1, Dr)

    scale = (Dh + Dr) ** -0.5
    scores = (jnp.einsum("bshd,bthd->bhst", Q, K)
              + jnp.einsum("bshd,bthd->bhst", Qr,
                           jnp.broadcast_to(Kr, (B, S, H, Dr)))) * scale
    m = scores.max(-1, keepdims=True)
    P = jnp.exp(scores - m)
    P = P / P.sum(-1, keepdims=True)
    O = jnp.einsum("bhst,bthd->bshd", P, V).reshape(B, S, H * Dh)
    return (O @ Wo).astype(jnp.float32)


# device time: 589480 ns/iter; 1.6910x vs baseline; 1.6910x over previous
import jax
import jax.numpy as jnp
from jax import lax
from jax.experimental import pallas as pl
from jax.experimental.pallas import tpu as pltpu

N_DEV = 16
B, S, H, Dh, Dr = 4, 256, 32, 128, 64
D = 4096
HB = D // N_DEV
NSLOTS = 2
MESH = pl.DeviceIdType.MESH


def _ring_rs_kv(k, v):

    def chunk(ref, idx):
        return ref.at[:, :, pl.ds(idx * HB, HB)]

    def body(k_hbm, v_hbm, ko_ref, vo_ref, kbuf, vbuf, krecv, vrecv,
             ks_sem, kr_sem, vs_sem, vr_sem, kcredit, vcredit,
             kcopy, vcopy):
        p = lax.axis_index("i")
        left = (p - 1) % N_DEV
        right = (p + 1) % N_DEV

        cpk = pltpu.make_async_copy(k_hbm, kbuf, kcopy)
        cpv = pltpu.make_async_copy(v_hbm, vbuf, vcopy)
        cpk.start()
        cpv.start()

        barrier_sem = pltpu.get_barrier_semaphore()
        for nbr in (left, right):
            pl.semaphore_signal(barrier_sem, inc=1, device_id=(nbr,),
                                device_id_type=MESH)
        pl.semaphore_wait(barrier_sem, 2)
        cpk.wait()
        cpv.wait()

        for s in range(N_DEV - 1):
            slot = s % NSLOTS
            k_send = (p - s) % N_DEV
            k_recv = (p - s - 1) % N_DEV
            v_send = (p + s + 2) % N_DEV
            v_recv = (p + s + 3) % N_DEV
            if s >= NSLOTS:
                pl.semaphore_wait(kcredit, 1)
                pl.semaphore_wait(vcredit, 1)
            krdma = pltpu.make_async_remote_copy(
                src_ref=chunk(kbuf, k_send), dst_ref=krecv.at[slot],
                send_sem=ks_sem.at[slot], recv_sem=kr_sem.at[slot],
                device_id=(right,), device_id_type=MESH)
            vrdma = pltpu.make_async_remote_copy(
                src_ref=chunk(vbuf, v_send), dst_ref=vrecv.at[slot],
                send_sem=vs_sem.at[slot], recv_sem=vr_sem.at[slot],
                device_id=(left,), device_id_type=MESH)
            krdma.start()
            vrdma.start()
            krdma.wait()
            vrdma.wait()
            kbuf[:, :, pl.ds(k_recv * HB, HB)] += krecv[slot]
            vbuf[:, :, pl.ds(v_recv * HB, HB)] += vrecv[slot]
            if s < (N_DEV - 1) - NSLOTS:
                pl.semaphore_signal(kcredit, inc=1, device_id=(left,),
                                    device_id_type=MESH)
                pl.semaphore_signal(vcredit, inc=1, device_id=(right,),
                                    device_id_type=MESH)

        own = (p + 1) % N_DEV
        ko_ref[...] = kbuf[:, :, pl.ds(own * HB, HB)]
        vo_ref[...] = vbuf[:, :, pl.ds(own * HB, HB)]

    return pl.pallas_call(
        body,
        out_shape=[jax.ShapeDtypeStruct((B, S, HB), k.dtype),
                   jax.ShapeDtypeStruct((B, S, HB), k.dtype)],
        in_specs=[pl.BlockSpec(memory_space=pltpu.MemorySpace.HBM)] * 2,
        out_specs=[pl.BlockSpec(memory_space=pltpu.VMEM)] * 2,
        scratch_shapes=[
            pltpu.VMEM((B, S, D), k.dtype),
            pltpu.VMEM((B, S, D), k.dtype),
            pltpu.VMEM((NSLOTS, B, S, HB), k.dtype),
            pltpu.VMEM((NSLOTS, B, S, HB), k.dtype),
            pltpu.SemaphoreType.DMA((NSLOTS,)),
            pltpu.SemaphoreType.DMA((NSLOTS,)),
            pltpu.SemaphoreType.DMA((NSLOTS,)),
            pltpu.SemaphoreType.DMA((NSLOTS,)),
            pltpu.SemaphoreType.REGULAR,
            pltpu.SemaphoreType.REGULAR,
            pltpu.SemaphoreType.DMA,
            pltpu.SemaphoreType.DMA,
        ],
        compiler_params=pltpu.CompilerParams(
            collective_id=0, vmem_limit_bytes=63 * 1024 * 1024),
    )(k, v)


def _ring_ar_out(y):
    half = N_DEV // 2
    CW = D // 2 // N_DEV

    def fchunk(ref, idx):
        return ref.at[:, :, pl.ds(idx * CW, CW)]

    def rchunk(ref, idx):
        return ref.at[:, :, pl.ds(D // 2 + idx * CW, CW)]

    def body(y_hbm, out_ref, frecv, rrecv,
             frs_s, frs_r, fag_s, fag_r, rrs_s, rrs_r, rag_s, rag_r,
             fcred_rs, fcred_ag, rcred_rs, rcred_ag, copy_sem):
        p = lax.axis_index("i")
        left = (p - 1) % N_DEV
        right = (p + 1) % N_DEV

        cp = pltpu.make_async_copy(y_hbm, out_ref, copy_sem)
        cp.start()

        barrier_sem = pltpu.get_barrier_semaphore()
        for nbr in (left, right):
            pl.semaphore_signal(barrier_sem, inc=1, device_id=(nbr,),
                                device_id_type=MESH)
        pl.semaphore_wait(barrier_sem, 2)
        cp.wait()

        for s in range(N_DEV - 1):
            slot = s % NSLOTS
            f_send = (p - s) % N_DEV
            f_recv = (p - s - 1) % N_DEV
            r_send = (p + s) % N_DEV
            r_recv = (p + s + 1) % N_DEV
            if s >= NSLOTS:
                pl.semaphore_wait(fcred_rs, 1)
                pl.semaphore_wait(rcred_rs, 1)
            frdma = pltpu.make_async_remote_copy(
                src_ref=fchunk(out_ref, f_send), dst_ref=frecv.at[slot],
                send_sem=frs_s.at[slot], recv_sem=frs_r.at[slot],
                device_id=(right,), device_id_type=MESH)
            rrdma = pltpu.make_async_remote_copy(
                src_ref=rchunk(out_ref, r_send), dst_ref=rrecv.at[slot],
                send_sem=rrs_s.at[slot], recv_sem=rrs_r.at[slot],
                device_id=(left,), device_id_type=MESH)
            frdma.start()
            rrdma.start()
            frdma.wait()
            rrdma.wait()
            out_ref[:, :, pl.ds(f_recv * CW, CW)] += frecv[slot]
            out_ref[:, :, pl.ds(D // 2 + r_recv * CW, CW)] += rrecv[slot]
            if s < (N_DEV - 1) - NSLOTS:
                pl.semaphore_signal(fcred_rs, inc=1, device_id=(left,),
                                    device_id_type=MESH)
                pl.semaphore_signal(rcred_rs, inc=1, device_id=(right,),
                                    device_id_type=MESH)

        for s in range(N_DEV - 1):
            slot = s % NSLOTS
            f_send = (p + 1 - s) % N_DEV
            r_send = (p - 1 + s) % N_DEV
            if s >= NSLOTS:
                pl.semaphore_wait(fcred_ag, 1)
                pl.semaphore_wait(rcred_ag, 1)
            frdma = pltpu.make_async_remote_copy(
                src_ref=fchunk(out_ref, f_send),
                dst_ref=fchunk(out_ref, f_send),
                send_sem=fag_s.at[slot], recv_sem=fag_r.at[slot],
                device_id=(right,), device_id_type=MESH)
            rrdma = pltpu.make_async_remote_copy(
                src_ref=rchunk(out_ref, r_send),
                dst_ref=rchunk(out_ref, r_send),
                send_sem=rag_s.at[slot], recv_sem=rag_r.at[slot],
                device_id=(left,), device_id_type=MESH)
            frdma.start()
            rrdma.start()
            frdma.wait()
            rrdma.wait()
            if s < (N_DEV - 1) - NSLOTS:
                pl.semaphore_signal(fcred_ag, inc=1, device_id=(left,),
                                    device_id_type=MESH)
                pl.semaphore_signal(rcred_ag, inc=1, device_id=(right,),
                                    device_id_type=MESH)

    return pl.pallas_call(
        body,
        out_shape=jax.ShapeDtypeStruct((B, S, D), y.dtype),
        in_specs=[pl.BlockSpec(memory_space=pltpu.MemorySpace.HBM)],
        out_specs=pl.BlockSpec(memory_space=pltpu.VMEM),
        scratch_shapes=[
            pltpu.VMEM((NSLOTS, B, S, CW), y.dtype),
            pltpu.VMEM((NSLOTS, B, S, CW), y.dtype),
            pltpu.SemaphoreType.DMA((NSLOTS,)),
            pltpu.SemaphoreType.DMA((NSLOTS,)),
            pltpu.SemaphoreType.DMA((NSLOTS,)),
            pltpu.SemaphoreType.DMA((NSLOTS,)),
            pltpu.SemaphoreType.DMA((NSLOTS,)),
            pltpu.SemaphoreType.DMA((NSLOTS,)),
            pltpu.SemaphoreType.DMA((NSLOTS,)),
            pltpu.SemaphoreType.DMA((NSLOTS,)),
            pltpu.SemaphoreType.REGULAR,
            pltpu.SemaphoreType.REGULAR,
            pltpu.SemaphoreType.REGULAR,
            pltpu.SemaphoreType.REGULAR,
            pltpu.SemaphoreType.DMA,
        ],
        compiler_params=pltpu.CompilerParams(
            collective_id=1, vmem_limit_bytes=63 * 1024 * 1024),
    )(y)


def kernel(x, Wdkv, Wuk, Wuv, Wq, Wqr, Wkr, Wo):
    c = x @ Wdkv
    Kp = c @ Wuk
    Vp = c @ Wuv
    k_own, v_own = _ring_rs_kv(Kp, Vp)

    p = lax.axis_index("i")
    o = (p + 1) % N_DEV
    nh = HB // Dh

    Ko = k_own.reshape(B, S, nh, Dh)
    Vo = v_own.reshape(B, S, nh, Dh)
    Wq_o = lax.dynamic_slice(Wq, (0, o * HB), (D, HB))
    Wqr_o = lax.dynamic_slice(Wqr, (0, o * nh * Dr), (D, nh * Dr))
    Qo = (x @ Wq_o).reshape(B, S, nh, Dh)
    Qro = (x @ Wqr_o).reshape(B, S, nh, Dr)
    Kr = (x @ Wkr).reshape(B, S, 1, Dr)

    scale = (Dh + Dr) ** -0.5
    scores = (jnp.einsum("bshd,bthd->bhst", Qo, Ko)
              + jnp.einsum("bshd,bthd->bhst", Qro,
                           jnp.broadcast_to(Kr, (B, S, nh, Dr)))) * scale
    m = scores.max(-1, keepdims=True)
    P = jnp.exp(scores - m)
    P = P / P.sum(-1, keepdims=True)
    O = jnp.einsum("bhst,bthd->bshd", P, Vo).reshape(B, S, HB)

    Wo_o = lax.dynamic_slice(Wo, (o * HB, 0), (HB, D))
    y = O @ Wo_o
    return _ring_ar_out(y).astype(jnp.float32)


# device time: 451859 ns/iter; 2.2061x vs baseline; 1.3046x over previous
import jax
import jax.numpy as jnp
from jax import lax
from jax.experimental import pallas as pl
from jax.experimental.pallas import tpu as pltpu

N_DEV = 16
B, S, H, Dh, Dr = 4, 256, 32, 128, 64
D = 4096
HB = D // N_DEV
NSLOTS = 2
NSTEP = N_DEV - 1
MESH = pl.DeviceIdType.MESH


class _Chain:

    def __init__(self, tgt, peer, src_slice, dst_slice, recv, ssem, rsem,
                 credit, add_slice=None):
        self.tgt = tgt
        self.peer = peer
        self.src_slice = src_slice
        self.dst_slice = dst_slice
        self.add_slice = add_slice
        self.recv = recv
        self.ssem = ssem
        self.rsem = rsem
        self.credit = credit
        self.cur = None

    def make(self, s):
        slot = s % NSLOTS
        return pltpu.make_async_remote_copy(
            src_ref=self.src_slice(s), dst_ref=self.dst_slice(s),
            send_sem=self.ssem.at[slot], recv_sem=self.rsem.at[slot],
            device_id=(self.tgt,), device_id_type=MESH)

    def start(self, s):
        if s >= NSLOTS:
            pl.semaphore_wait(self.credit, 1)
        self.cur = self.make(s)
        self.cur.start()

    def finish(self, s):
        self.cur.wait()
        if self.add_slice is not None:
            slot = s % NSLOTS
            tgt = self.add_slice(s)
            tgt[...] = tgt[...] + self.recv[slot]
        if s < NSTEP - NSLOTS:
            pl.semaphore_signal(self.credit, inc=1, device_id=(self.peer,),
                                device_id_type=MESH)


def _run_chains(chains):
    for c in chains:
        c.start(0)
    for s in range(NSTEP):
        for c in chains:
            c.finish(s)
            if s + 1 < NSTEP:
                c.start(s + 1)


def _ring_rs_kv(k, v):
    CW = HB // 2

    def body(k_hbm, v_hbm, ko_ref, vo_ref, kbuf, vbuf,
             krecv_a, krecv_b, vrecv_a, vrecv_b,
             ks_a, kr_a, ks_b, kr_b, vs_a, vr_a, vs_b, vr_b,
             kcred_a, kcred_b, vcred_a, vcred_b, kcopy, vcopy):
        p = lax.axis_index("i")
        left = (p - 1) % N_DEV
        right = (p + 1) % N_DEV

        cpk = pltpu.make_async_copy(k_hbm, kbuf, kcopy)
        cpv = pltpu.make_async_copy(v_hbm, vbuf, vcopy)
        cpk.start()
        cpv.start()

        barrier_sem = pltpu.get_barrier_semaphore()
        for nbr in (left, right):
            pl.semaphore_signal(barrier_sem, inc=1, device_id=(nbr,),
                                device_id_type=MESH)
        pl.semaphore_wait(barrier_sem, 2)
        cpk.wait()
        cpv.wait()

        def strip(buf, idx, off):
            return buf.at[:, :, pl.ds(idx * HB + off, CW)]

        def mk_chain(buf, tgt, peer, send_idx, recv_idx, off, recv,
                     ssem, rsem, credit):
            return _Chain(
                tgt, peer,
                src_slice=lambda s: strip(buf, send_idx(s), off),
                dst_slice=lambda s: recv.at[s % NSLOTS],
                add_slice=lambda s: strip(buf, recv_idx(s), off),
                recv=recv, ssem=ssem, rsem=rsem, credit=credit)

        k_send = lambda s: (p - s) % N_DEV
        k_recv = lambda s: (p - s - 1) % N_DEV
        v_send = lambda s: (p + s + 2) % N_DEV
        v_recv = lambda s: (p + s + 3) % N_DEV

        chains = [
            mk_chain(kbuf, right, left, k_send, k_recv, 0, krecv_a,
                     ks_a, kr_a, kcred_a),
            mk_chain(vbuf, left, right, v_send, v_recv, 0, vrecv_a,
                     vs_a, vr_a, vcred_a),
            mk_chain(kbuf, right, left, k_send, k_recv, CW, krecv_b,
                     ks_b, kr_b, kcred_b),
            mk_chain(vbuf, left, right, v_send, v_recv, CW, vrecv_b,
                     vs_b, vr_b, vcred_b),
        ]
        _run_chains(chains)

        own = (p + 1) % N_DEV
        ko_ref[...] = kbuf[:, :, pl.ds(own * HB, HB)]
        vo_ref[...] = vbuf[:, :, pl.ds(own * HB, HB)]

    dma2 = pltpu.SemaphoreType.DMA((NSLOTS,))
    return pl.pallas_call(
        body,
        out_shape=[jax.ShapeDtypeStruct((B, S, HB), k.dtype),
                   jax.ShapeDtypeStruct((B, S, HB), k.dtype)],
        in_specs=[pl.BlockSpec(memory_space=pltpu.MemorySpace.HBM)] * 2,
        out_specs=[pl.BlockSpec(memory_space=pltpu.VMEM)] * 2,
        scratch_shapes=(
            [pltpu.VMEM((B, S, D), k.dtype)] * 2
            + [pltpu.VMEM((NSLOTS, B, S, CW), k.dtype)] * 4
            + [dma2] * 8
            + [pltpu.SemaphoreType.REGULAR] * 4
            + [pltpu.SemaphoreType.DMA] * 2
        ),
        compiler_params=pltpu.CompilerParams(
            collective_id=0, vmem_limit_bytes=63 * 1024 * 1024),
    )(k, v)


def _ring_ar_out(y):
    SC = S // N_DEV
    Q = D // 4

    def body(y_hbm, out_ref, r0, r1, r2, r3,
             rs_s0, rs_r0, rs_s1, rs_r1, rs_s2, rs_r2, rs_s3, rs_r3,
             ag_s0, ag_r0, ag_s1, ag_r1, ag_s2, ag_r2, ag_s3, ag_r3,
             rc0, rc1, rc2, rc3, ac0, ac1, ac2, ac3, copy_sem):
        p = lax.axis_index("i")
        left = (p - 1) % N_DEV
        right = (p + 1) % N_DEV

        cp = pltpu.make_async_copy(y_hbm, out_ref, copy_sem)
        cp.start()

        barrier_sem = pltpu.get_barrier_semaphore()
        for nbr in (left, right):
            pl.semaphore_signal(barrier_sem, inc=1, device_id=(nbr,),
                                device_id_type=MESH)
        pl.semaphore_wait(barrier_sem, 2)
        cp.wait()

        def strip(idx, q):
            return out_ref.at[:, pl.ds(idx * SC, SC), q * Q:(q + 1) * Q]

        f_rs_send = lambda s: (p - s) % N_DEV
        f_rs_recv = lambda s: (p - s - 1) % N_DEV
        r_rs_send = lambda s: (p + s) % N_DEV
        r_rs_recv = lambda s: (p + s + 1) % N_DEV
        f_ag_send = lambda s: (p + 1 - s) % N_DEV
        r_ag_send = lambda s: (p - 1 + s) % N_DEV

        def rs_chain(q, tgt, peer, send_idx, recv_idx, recv, ssem, rsem,
                     credit):
            return _Chain(
                tgt, peer,
                src_slice=lambda s: strip(send_idx(s), q),
                dst_slice=lambda s: recv.at[s % NSLOTS],
                add_slice=lambda s: strip(recv_idx(s), q),
                recv=recv, ssem=ssem, rsem=rsem, credit=credit)

        def ag_chain(q, tgt, peer, send_idx, ssem, rsem, credit):
            return _Chain(
                tgt, peer,
                src_slice=lambda s: strip(send_idx(s), q),
                dst_slice=lambda s: strip(send_idx(s), q),
                add_slice=None,
                recv=None, ssem=ssem, rsem=rsem, credit=credit)

        _run_chains([
            rs_chain(0, right, left, f_rs_send, f_rs_recv, r0, rs_s0, rs_r0, rc0),
            rs_chain(2, left, right, r_rs_send, r_rs_recv, r2, rs_s2, rs_r2, rc2),
            rs_chain(1, right, left, f_rs_send, f_rs_recv, r1, rs_s1, rs_r1, rc1),
            rs_chain(3, left, right, r_rs_send, r_rs_recv, r3, rs_s3, rs_r3, rc3),
        ])
        _run_chains([
            ag_chain(0, right, left, f_ag_send, ag_s0, ag_r0, ac0),
            ag_chain(2, left, right, r_ag_send, ag_s2, ag_r2, ac2),
            ag_chain(1, right, left, f_ag_send, ag_s1, ag_r1, ac1),
            ag_chain(3, left, right, r_ag_send, ag_s3, ag_r3, ac3),
        ])

    dma2 = pltpu.SemaphoreType.DMA((NSLOTS,))
    return pl.pallas_call(
        body,
        out_shape=jax.ShapeDtypeStruct((B, S, D), y.dtype),
        in_specs=[pl.BlockSpec(memory_space=pltpu.MemorySpace.HBM)],
        out_specs=pl.BlockSpec(memory_space=pltpu.VMEM),
        scratch_shapes=(
            [pltpu.VMEM((NSLOTS, B, SC, Q), y.dtype)] * 4
            + [dma2] * 16
            + [pltpu.SemaphoreType.REGULAR] * 8
            + [pltpu.SemaphoreType.DMA]
        ),
        compiler_params=pltpu.CompilerParams(
            collective_id=1, vmem_limit_bytes=63 * 1024 * 1024),
    )(y)


def kernel(x, Wdkv, Wuk, Wuv, Wq, Wqr, Wkr, Wo):
    c = x @ Wdkv
    Kp = c @ Wuk
    Vp = c @ Wuv
    k_own, v_own = _ring_rs_kv(Kp, Vp)

    p = lax.axis_index("i")
    o = (p + 1) % N_DEV
    nh = HB // Dh

    Ko = k_own.reshape(B, S, nh, Dh)
    Vo = v_own.reshape(B, S, nh, Dh)
    Wq_o = lax.dynamic_slice(Wq, (0, o * HB), (D, HB))
    Wqr_o = lax.dynamic_slice(Wqr, (0, o * nh * Dr), (D, nh * Dr))
    Qo = (x @ Wq_o).reshape(B, S, nh, Dh)
    Qro = (x @ Wqr_o).reshape(B, S, nh, Dr)
    Kr = (x @ Wkr).reshape(B, S, 1, Dr)

    scale = (Dh + Dr) ** -0.5
    scores = (jnp.einsum("bshd,bthd->bhst", Qo, Ko)
              + jnp.einsum("bshd,bthd->bhst", Qro,
                           jnp.broadcast_to(Kr, (B, S, nh, Dr)))) * scale
    m = scores.max(-1, keepdims=True)
    P = jnp.exp(scores - m)
    P = P / P.sum(-1, keepdims=True)
    O = jnp.einsum("bhst,bthd->bshd", P, Vo).reshape(B, S, HB)

    Wo_o = lax.dynamic_slice(Wo, (o * HB, 0), (HB, D))
    y = O @ Wo_o
    return _ring_ar_out(y).astype(jnp.float32)


# device time: 280169 ns/iter; 3.5579x vs baseline; 1.6128x over previous
import jax
import jax.numpy as jnp
from jax import lax
from jax.experimental import pallas as pl
from jax.experimental.pallas import tpu as pltpu

N_DEV = 16
B, S, H, Dh, Dr = 4, 256, 32, 128, 64
D = 4096
HB = D // N_DEV
NSLOTS = 2
NSTEP = N_DEV - 1
MESH = pl.DeviceIdType.MESH


class _Chain:

    def __init__(self, tgt, peer, src_slice, dst_slice, recv, ssem, rsem,
                 credit, add_slice=None):
        self.tgt = tgt
        self.peer = peer
        self.src_slice = src_slice
        self.dst_slice = dst_slice
        self.add_slice = add_slice
        self.recv = recv
        self.ssem = ssem
        self.rsem = rsem
        self.credit = credit
        self.cur = None

    def make(self, s):
        slot = s % NSLOTS
        return pltpu.make_async_remote_copy(
            src_ref=self.src_slice(s), dst_ref=self.dst_slice(s),
            send_sem=self.ssem.at[slot], recv_sem=self.rsem.at[slot],
            device_id=(self.tgt,), device_id_type=MESH)

    def start(self, s):
        if s >= NSLOTS:
            pl.semaphore_wait(self.credit, 1)
        self.cur = self.make(s)
        self.cur.start()

    def finish(self, s):
        self.cur.wait()
        if self.add_slice is not None:
            slot = s % NSLOTS
            tgt = self.add_slice(s)
            tgt[...] = tgt[...] + self.recv[slot]
        if s < NSTEP - NSLOTS:
            pl.semaphore_signal(self.credit, inc=1, device_id=(self.peer,),
                                device_id_type=MESH)


def _run_chains(chains):
    for c in chains:
        c.start(0)
    for s in range(NSTEP):
        for c in chains:
            c.finish(s)
            if s + 1 < NSTEP:
                c.start(s + 1)


def _ring_rs_kv(k, v):
    CW = HB // 2

    def body(k_hbm, v_hbm, ko_ref, vo_ref, kbuf, vbuf,
             krecv_a, krecv_b, vrecv_a, vrecv_b,
             ks_a, kr_a, ks_b, kr_b, vs_a, vr_a, vs_b, vr_b,
             kcred_a, kcred_b, vcred_a, vcred_b, kcopy, vcopy):
        p = lax.axis_index("i")
        left = (p - 1) % N_DEV
        right = (p + 1) % N_DEV

        cpk = pltpu.make_async_copy(k_hbm, kbuf, kcopy)
        cpv = pltpu.make_async_copy(v_hbm, vbuf, vcopy)
        cpk.start()
        cpv.start()

        barrier_sem = pltpu.get_barrier_semaphore()
        for nbr in (left, right):
            pl.semaphore_signal(barrier_sem, inc=1, device_id=(nbr,),
                                device_id_type=MESH)
        pl.semaphore_wait(barrier_sem, 2)
        cpk.wait()
        cpv.wait()

        def strip(buf, idx, off):
            return buf.at[:, :, pl.ds(idx * HB + off, CW)]

        def mk_chain(buf, tgt, peer, send_idx, recv_idx, off, recv,
                     ssem, rsem, credit):
            return _Chain(
                tgt, peer,
                src_slice=lambda s: strip(buf, send_idx(s), off),
                dst_slice=lambda s: recv.at[s % NSLOTS],
                add_slice=lambda s: strip(buf, recv_idx(s), off),
                recv=recv, ssem=ssem, rsem=rsem, credit=credit)

        k_send = lambda s: (p - s) % N_DEV
        k_recv = lambda s: (p - s - 1) % N_DEV
        v_send = lambda s: (p + s + 2) % N_DEV
        v_recv = lambda s: (p + s + 3) % N_DEV

        chains = [
            mk_chain(kbuf, right, left, k_send, k_recv, 0, krecv_a,
                     ks_a, kr_a, kcred_a),
            mk_chain(vbuf, left, right, v_send, v_recv, 0, vrecv_a,
                     vs_a, vr_a, vcred_a),
            mk_chain(kbuf, right, left, k_send, k_recv, CW, krecv_b,
                     ks_b, kr_b, kcred_b),
            mk_chain(vbuf, left, right, v_send, v_recv, CW, vrecv_b,
                     vs_b, vr_b, vcred_b),
        ]
        _run_chains(chains)

        own = (p + 1) % N_DEV
        ko_ref[...] = kbuf[:, :, pl.ds(own * HB, HB)]
        vo_ref[...] = vbuf[:, :, pl.ds(own * HB, HB)]

    dma2 = pltpu.SemaphoreType.DMA((NSLOTS,))
    return pl.pallas_call(
        body,
        out_shape=[jax.ShapeDtypeStruct((B, S, HB), k.dtype),
                   jax.ShapeDtypeStruct((B, S, HB), k.dtype)],
        in_specs=[pl.BlockSpec(memory_space=pltpu.MemorySpace.HBM)] * 2,
        out_specs=[pl.BlockSpec(memory_space=pltpu.VMEM)] * 2,
        scratch_shapes=(
            [pltpu.VMEM((B, S, D), k.dtype)] * 2
            + [pltpu.VMEM((NSLOTS, B, S, CW), k.dtype)] * 4
            + [dma2] * 8
            + [pltpu.SemaphoreType.REGULAR] * 4
            + [pltpu.SemaphoreType.DMA] * 2
        ),
        compiler_params=pltpu.CompilerParams(
            collective_id=0, vmem_limit_bytes=63 * 1024 * 1024),
    )(k, v)


def _ring_ar_out(y):
    SC = S // N_DEV
    Q = D // 4

    def body(y_hbm, out_ref, r0, r1, r2, r3,
             rs_s0, rs_r0, rs_s1, rs_r1, rs_s2, rs_r2, rs_s3, rs_r3,
             ag_s0, ag_r0, ag_s1, ag_r1, ag_s2, ag_r2, ag_s3, ag_r3,
             rc0, rc1, rc2, rc3, ac0, ac1, ac2, ac3, copy_sem):
        p = lax.axis_index("i")
        left = (p - 1) % N_DEV
        right = (p + 1) % N_DEV

        cp = pltpu.make_async_copy(y_hbm, out_ref, copy_sem)
        cp.start()

        barrier_sem = pltpu.get_barrier_semaphore()
        for nbr in (left, right):
            pl.semaphore_signal(barrier_sem, inc=1, device_id=(nbr,),
                                device_id_type=MESH)
        pl.semaphore_wait(barrier_sem, 2)
        cp.wait()

        def strip(idx, q):
            return out_ref.at[:, pl.ds(idx * SC, SC), q * Q:(q + 1) * Q]

        f_rs_send = lambda s: (p - s) % N_DEV
        f_rs_recv = lambda s: (p - s - 1) % N_DEV
        r_rs_send = lambda s: (p + s) % N_DEV
        r_rs_recv = lambda s: (p + s + 1) % N_DEV
        f_ag_send = lambda s: (p + 1 - s) % N_DEV
        r_ag_send = lambda s: (p - 1 + s) % N_DEV

        def rs_chain(q, tgt, peer, send_idx, recv_idx, recv, ssem, rsem,
                     credit):
            return _Chain(
                tgt, peer,
                src_slice=lambda s: strip(send_idx(s), q),
                dst_slice=lambda s: recv.at[s % NSLOTS],
                add_slice=lambda s: strip(recv_idx(s), q),
                recv=recv, ssem=ssem, rsem=rsem, credit=credit)

        def ag_chain(q, tgt, peer, send_idx, ssem, rsem, credit):
            return _Chain(
                tgt, peer,
                src_slice=lambda s: strip(send_idx(s), q),
                dst_slice=lambda s: strip(send_idx(s), q),
                add_slice=None,
                recv=None, ssem=ssem, rsem=rsem, credit=credit)

        _run_chains([
            rs_chain(0, right, left, f_rs_send, f_rs_recv, r0, rs_s0, rs_r0, rc0),
            rs_chain(2, left, right, r_rs_send, r_rs_recv, r2, rs_s2, rs_r2, rc2),
            rs_chain(1, right, left, f_rs_send, f_rs_recv, r1, rs_s1, rs_r1, rc1),
            rs_chain(3, left, right, r_rs_send, r_rs_recv, r3, rs_s3, rs_r3, rc3),
        ])
        _run_chains([
            ag_chain(0, right, left, f_ag_send, ag_s0, ag_r0, ac0),
            ag_chain(2, left, right, r_ag_send, ag_s2, ag_r2, ac2),
            ag_chain(1, right, left, f_ag_send, ag_s1, ag_r1, ac1),
            ag_chain(3, left, right, r_ag_send, ag_s3, ag_r3, ac3),
        ])

    dma2 = pltpu.SemaphoreType.DMA((NSLOTS,))
    return pl.pallas_call(
        body,
        out_shape=jax.ShapeDtypeStruct((B, S, D), y.dtype),
        in_specs=[pl.BlockSpec(memory_space=pltpu.MemorySpace.HBM)],
        out_specs=pl.BlockSpec(memory_space=pltpu.VMEM),
        scratch_shapes=(
            [pltpu.VMEM((NSLOTS, B, SC, Q), y.dtype)] * 4
            + [dma2] * 16
            + [pltpu.SemaphoreType.REGULAR] * 8
            + [pltpu.SemaphoreType.DMA]
        ),
        compiler_params=pltpu.CompilerParams(
            collective_id=1, vmem_limit_bytes=63 * 1024 * 1024),
    )(y)


def kernel(x, Wdkv, Wuk, Wuv, Wq, Wqr, Wkr, Wo):
    c = x @ Wdkv
    Kp = c @ Wuk
    Vp = c @ Wuv
    k_own, v_own = _ring_rs_kv(Kp.astype(jnp.bfloat16),
                               Vp.astype(jnp.bfloat16))
    k_own = k_own.astype(jnp.float32)
    v_own = v_own.astype(jnp.float32)

    p = lax.axis_index("i")
    o = (p + 1) % N_DEV
    nh = HB // Dh

    Ko = k_own.reshape(B, S, nh, Dh)
    Vo = v_own.reshape(B, S, nh, Dh)
    Wq_o = lax.dynamic_slice(Wq, (0, o * HB), (D, HB))
    Wqr_o = lax.dynamic_slice(Wqr, (0, o * nh * Dr), (D, nh * Dr))
    Qo = (x @ Wq_o).reshape(B, S, nh, Dh)
    Qro = (x @ Wqr_o).reshape(B, S, nh, Dr)
    Kr = (x @ Wkr).reshape(B, S, 1, Dr)

    scale = (Dh + Dr) ** -0.5
    scores = (jnp.einsum("bshd,bthd->bhst", Qo, Ko)
              + jnp.einsum("bshd,bthd->bhst", Qro,
                           jnp.broadcast_to(Kr, (B, S, nh, Dr)))) * scale
    m = scores.max(-1, keepdims=True)
    P = jnp.exp(scores - m)
    P = P / P.sum(-1, keepdims=True)
    O = jnp.einsum("bhst,bthd->bshd", P, Vo).reshape(B, S, HB)

    Wo_o = lax.dynamic_slice(Wo, (o * HB, 0), (HB, D))
    y = O @ Wo_o
    return _ring_ar_out(y.astype(jnp.bfloat16)).astype(jnp.float32)


# device time: 274235 ns/iter; 3.6349x vs baseline; 1.0216x over previous
import jax
import jax.numpy as jnp
from jax import lax
from jax.experimental import pallas as pl
from jax.experimental.pallas import tpu as pltpu

N_DEV = 16
B, S, H, Dh, Dr = 4, 256, 32, 128, 64
D = 4096
HB = D // N_DEV
NSLOTS = 2
NSTEP = N_DEV - 1
MESH = pl.DeviceIdType.MESH


class _Chain:

    def __init__(self, tgt, peer, src_slice, dst_slice, recv, ssem, rsem,
                 credit, add_slice=None):
        self.tgt = tgt
        self.peer = peer
        self.src_slice = src_slice
        self.dst_slice = dst_slice
        self.add_slice = add_slice
        self.recv = recv
        self.ssem = ssem
        self.rsem = rsem
        self.credit = credit
        self.cur = None

    def make(self, s):
        slot = s % NSLOTS
        return pltpu.make_async_remote_copy(
            src_ref=self.src_slice(s), dst_ref=self.dst_slice(s),
            send_sem=self.ssem.at[slot], recv_sem=self.rsem.at[slot],
            device_id=(self.tgt,), device_id_type=MESH)

    def start(self, s):
        if s >= NSLOTS:
            pl.semaphore_wait(self.credit, 1)
        self.cur = self.make(s)
        self.cur.start()

    def finish(self, s):
        self.cur.wait()
        if self.add_slice is not None:
            slot = s % NSLOTS
            tgt = self.add_slice(s)
            tgt[...] = tgt[...] + self.recv[slot]
        if s < NSTEP - NSLOTS:
            pl.semaphore_signal(self.credit, inc=1, device_id=(self.peer,),
                                device_id_type=MESH)


def _run_chains(chains):
    for c in chains:
        c.start(0)
    for s in range(NSTEP):
        for c in chains:
            c.finish(s)
            if s + 1 < NSTEP:
                c.start(s + 1)


def _ring_rs_kv(k, v):
    CW = HB // 2

    def body(k_hbm, v_hbm, ko_ref, vo_ref, kbuf, vbuf,
             krecv_a, krecv_b, vrecv_a, vrecv_b,
             ks_a, kr_a, ks_b, kr_b, vs_a, vr_a, vs_b, vr_b,
             kcred_a, kcred_b, vcred_a, vcred_b, kcopy, vcopy):
        p = lax.axis_index("i")
        left = (p - 1) % N_DEV
        right = (p + 1) % N_DEV

        cpk = pltpu.make_async_copy(k_hbm, kbuf, kcopy)
        cpv = pltpu.make_async_copy(v_hbm, vbuf, vcopy)
        cpk.start()
        cpv.start()

        barrier_sem = pltpu.get_barrier_semaphore()
        for nbr in (left, right):
            pl.semaphore_signal(barrier_sem, inc=1, device_id=(nbr,),
                                device_id_type=MESH)
        pl.semaphore_wait(barrier_sem, 2)
        cpk.wait()
        cpv.wait()

        def strip(buf, idx, off):
            return buf.at[:, :, pl.ds(idx * HB + off, CW)]

        def mk_chain(buf, tgt, peer, send_idx, recv_idx, off, recv,
                     ssem, rsem, credit):
            return _Chain(
                tgt, peer,
                src_slice=lambda s: strip(buf, send_idx(s), off),
                dst_slice=lambda s: recv.at[s % NSLOTS],
                add_slice=lambda s: strip(buf, recv_idx(s), off),
                recv=recv, ssem=ssem, rsem=rsem, credit=credit)

        k_send = lambda s: (p - s) % N_DEV
        k_recv = lambda s: (p - s - 1) % N_DEV
        v_send = lambda s: (p + s + 2) % N_DEV
        v_recv = lambda s: (p + s + 3) % N_DEV

        chains = [
            mk_chain(kbuf, right, left, k_send, k_recv, 0, krecv_a,
                     ks_a, kr_a, kcred_a),
            mk_chain(vbuf, left, right, v_send, v_recv, 0, vrecv_a,
                     vs_a, vr_a, vcred_a),
            mk_chain(kbuf, right, left, k_send, k_recv, CW, krecv_b,
                     ks_b, kr_b, kcred_b),
            mk_chain(vbuf, left, right, v_send, v_recv, CW, vrecv_b,
                     vs_b, vr_b, vcred_b),
        ]
        _run_chains(chains)

        own = (p + 1) % N_DEV
        ko_ref[...] = kbuf[:, :, pl.ds(own * HB, HB)]
        vo_ref[...] = vbuf[:, :, pl.ds(own * HB, HB)]

    dma2 = pltpu.SemaphoreType.DMA((NSLOTS,))
    return pl.pallas_call(
        body,
        out_shape=[jax.ShapeDtypeStruct((B, S, HB), k.dtype),
                   jax.ShapeDtypeStruct((B, S, HB), k.dtype)],
        in_specs=[pl.BlockSpec(memory_space=pltpu.MemorySpace.HBM)] * 2,
        out_specs=[pl.BlockSpec(memory_space=pltpu.VMEM)] * 2,
        scratch_shapes=(
            [pltpu.VMEM((B, S, D), k.dtype)] * 2
            + [pltpu.VMEM((NSLOTS, B, S, CW), k.dtype)] * 4
            + [dma2] * 8
            + [pltpu.SemaphoreType.REGULAR] * 4
            + [pltpu.SemaphoreType.DMA] * 2
        ),
        compiler_params=pltpu.CompilerParams(
            collective_id=0, vmem_limit_bytes=63 * 1024 * 1024),
    )(k, v)


N_SUB = 8


def _ring_ar_out(y):
    SC = S // N_DEV
    Q = D // N_SUB

    def body(y_hbm, out_ref, *scr):
        recvs = scr[0:N_SUB]
        rs_s = scr[N_SUB:2 * N_SUB]
        rs_r = scr[2 * N_SUB:3 * N_SUB]
        ag_s = scr[3 * N_SUB:4 * N_SUB]
        ag_r = scr[4 * N_SUB:5 * N_SUB]
        rcred = scr[5 * N_SUB:6 * N_SUB]
        acred = scr[6 * N_SUB:7 * N_SUB]
        copy_sem = scr[7 * N_SUB]

        p = lax.axis_index("i")
        left = (p - 1) % N_DEV
        right = (p + 1) % N_DEV

        cp = pltpu.make_async_copy(y_hbm, out_ref, copy_sem)
        cp.start()

        barrier_sem = pltpu.get_barrier_semaphore()
        for nbr in (left, right):
            pl.semaphore_signal(barrier_sem, inc=1, device_id=(nbr,),
                                device_id_type=MESH)
        pl.semaphore_wait(barrier_sem, 2)
        cp.wait()

        def strip(idx, q):
            return out_ref.at[:, pl.ds(idx * SC, SC), q * Q:(q + 1) * Q]

        f_rs_send = lambda s: (p - s) % N_DEV
        f_rs_recv = lambda s: (p - s - 1) % N_DEV
        r_rs_send = lambda s: (p + s) % N_DEV
        r_rs_recv = lambda s: (p + s + 1) % N_DEV
        f_ag_send = lambda s: (p + 1 - s) % N_DEV
        r_ag_send = lambda s: (p - 1 + s) % N_DEV

        def rs_chain(q):
            fwd = q < N_SUB // 2
            return _Chain(
                right if fwd else left, left if fwd else right,
                src_slice=(lambda s, q=q, f=fwd:
                           strip((f_rs_send if f else r_rs_send)(s), q)),
                dst_slice=lambda s, q=q: recvs[q].at[s % NSLOTS],
                add_slice=(lambda s, q=q, f=fwd:
                           strip((f_rs_recv if f else r_rs_recv)(s), q)),
                recv=recvs[q], ssem=rs_s[q], rsem=rs_r[q], credit=rcred[q])

        def ag_chain(q):
            fwd = q < N_SUB // 2
            send = f_ag_send if fwd else r_ag_send
            return _Chain(
                right if fwd else left, left if fwd else right,
                src_slice=lambda s, q=q, send=send: strip(send(s), q),
                dst_slice=lambda s, q=q, send=send: strip(send(s), q),
                add_slice=None,
                recv=None, ssem=ag_s[q], rsem=ag_r[q], credit=acred[q])

        order = [0, 4, 1, 5, 2, 6, 3, 7]
        _run_chains([rs_chain(q) for q in order])
        _run_chains([ag_chain(q) for q in order])

    dma2 = pltpu.SemaphoreType.DMA((NSLOTS,))
    return pl.pallas_call(
        body,
        out_shape=jax.ShapeDtypeStruct((B, S, D), y.dtype),
        in_specs=[pl.BlockSpec(memory_space=pltpu.MemorySpace.HBM)],
        out_specs=pl.BlockSpec(memory_space=pltpu.VMEM),
        scratch_shapes=(
            [pltpu.VMEM((NSLOTS, B, SC, Q), y.dtype)] * N_SUB
            + [dma2] * (4 * N_SUB)
            + [pltpu.SemaphoreType.REGULAR] * (2 * N_SUB)
            + [pltpu.SemaphoreType.DMA]
        ),
        compiler_params=pltpu.CompilerParams(
            collective_id=1, vmem_limit_bytes=63 * 1024 * 1024),
    )(y)


def kernel(x, Wdkv, Wuk, Wuv, Wq, Wqr, Wkr, Wo):
    c = x @ Wdkv
    Kp = jnp.matmul(c, Wuk, preferred_element_type=jnp.bfloat16)
    Vp = jnp.matmul(c, Wuv, preferred_element_type=jnp.bfloat16)
    k_own, v_own = _ring_rs_kv(Kp, Vp)
    k_own = k_own.astype(jnp.float32)
    v_own = v_own.astype(jnp.float32)

    p = lax.axis_index("i")
    o = (p + 1) % N_DEV
    nh = HB // Dh

    Ko = k_own.reshape(B, S, nh, Dh)
    Vo = v_own.reshape(B, S, nh, Dh)
    Wq_o = lax.dynamic_slice(Wq, (0, o * HB), (D, HB))
    Wqr_o = lax.dynamic_slice(Wqr, (0, o * nh * Dr), (D, nh * Dr))
    Qo = (x @ Wq_o).reshape(B, S, nh, Dh)
    Qro = (x @ Wqr_o).reshape(B, S, nh, Dr)
    Kr = (x @ Wkr).reshape(B, S, 1, Dr)

    scale = (Dh + Dr) ** -0.5
    scores = (jnp.einsum("bshd,bthd->bhst", Qo, Ko)
              + jnp.einsum("bshd,bthd->bhst", Qro,
                           jnp.broadcast_to(Kr, (B, S, nh, Dr)))) * scale
    m = scores.max(-1, keepdims=True)
    P = jnp.exp(scores - m)
    P = P / P.sum(-1, keepdims=True)
    O = jnp.einsum("bhst,bthd->bshd", P, Vo).reshape(B, S, HB)

    Wo_o = lax.dynamic_slice(Wo, (o * HB, 0), (HB, D))
    y = jnp.matmul(O, Wo_o, preferred_element_type=jnp.bfloat16)
    return _ring_ar_out(y).astype(jnp.float32)
